# baseline (speedup 1.0000x reference)
import functools

import jax
import jax.numpy as jnp
from jax.experimental import pallas as pl
from jax.experimental.pallas import tpu as pltpu

B, L, D = 4, 2048, 768
E, K, H = 8, 2, 16
NSPLIT = 2
LS = L // NSPLIT


def _moe_kernel(*refs):
    x_refs = refs[:NSPLIT]
    (w_gate_ref, expert_w_ref, expert_b_ref, shared_w_ref, shared_b_ref,
     out_ref, loss_ref, gates_acc) = refs[NSPLIT:]
    b = pl.program_id(0)
    nb = pl.num_programs(0)

    gx = x_refs[0][0].sum(axis=0, keepdims=True)
    for r in x_refs[1:]:
        gx = gx + r[0].sum(axis=0, keepdims=True)
    gx = gx * (1.0 / L)
    logits = jnp.dot(gx, w_gate_ref[...],
                     preferred_element_type=jnp.float32)

    lane = jax.lax.broadcasted_iota(jnp.int32, (1, E), 1)
    m1 = jnp.max(logits)
    i1 = jnp.min(jnp.where(logits == m1, lane, E))
    mask1 = lane == i1
    l2 = jnp.where(mask1, -jnp.inf, logits)
    m2 = jnp.max(l2)
    i2 = jnp.min(jnp.where(l2 == m2, lane, E))
    mask2 = lane == i2
    t = jnp.exp(m2 - m1)
    g1 = 1.0 / (1.0 + t)
    g2 = t / (1.0 + t)
    gates_row = jnp.where(mask1, g1, 0.0) + jnp.where(mask2, g2, 0.0)

    m_w = shared_w_ref[...]
    bias = shared_b_ref[...]
    for e in range(E):
        ge = jnp.sum(jnp.where(lane == e, gates_row, 0.0))
        m_w = m_w + ge * expert_w_ref[e]
        bias = bias + ge * expert_b_ref[e][None, :]
    for i, r in enumerate(x_refs):
        yb = jax.lax.dot_general(
            r[0], m_w,
            (((1,), (0,)), ((), ())),
            precision=jax.lax.Precision.HIGHEST,
            preferred_element_type=jnp.float32)
        out_ref[0, i * LS:(i + 1) * LS, :] = yb + bias

    row = jax.lax.broadcasted_iota(jnp.int32, (B, E), 0)

    @pl.when(b == 0)
    def _():
        gates_acc[...] = jnp.zeros((B, E), jnp.float32)

    gates_acc[...] = jnp.where(row == b, gates_row, gates_acc[...])

    @pl.when(b == nb - 1)
    def _():
        gates_all = gates_acc[...]
        eps = 1e-10

        def cv2(v):
            mean = jnp.sum(v) * (1.0 / E)
            var = jnp.sum((v - mean) ** 2) * (1.0 / (E - 1))
            return var / (mean * mean + eps)

        importance = jnp.sum(gates_all, axis=0, keepdims=True)
        load = jnp.sum((gates_all > 0).astype(jnp.float32), axis=0,
                       keepdims=True)
        loss_ref[...] = jnp.full((1, 1), (cv2(importance) + cv2(load)) * 1e-2,
                                 jnp.float32)


@functools.partial(jax.jit, static_argnames=("interpret",))
def kernel(x, w_gate, expert_w, expert_b, shared_w, shared_b,
           interpret=False):
    x_specs = [
        pl.BlockSpec((1, LS, D), functools.partial(lambda i, b: (b, i, 0), i))
        for i in range(NSPLIT)
    ]
    out, loss = pl.pallas_call(
        _moe_kernel,
        grid=(B,),
        in_specs=x_specs + [
            pl.BlockSpec((D, E), lambda b: (0, 0)),
            pl.BlockSpec((E, D, H), lambda b: (0, 0, 0)),
            pl.BlockSpec((E, H), lambda b: (0, 0)),
            pl.BlockSpec((D, H), lambda b: (0, 0)),
            pl.BlockSpec((1, H), lambda b: (0, 0)),
        ],
        out_specs=[
            pl.BlockSpec((1, L, H), lambda b: (b, 0, 0)),
            pl.BlockSpec((1, 1), lambda b: (0, 0)),
        ],
        out_shape=[
            jax.ShapeDtypeStruct((B, L, H), jnp.float32),
            jax.ShapeDtypeStruct((1, 1), jnp.float32),
        ],
        scratch_shapes=[pltpu.VMEM((B, E), jnp.float32)],
        interpret=interpret,
    )(*([x] * NSPLIT), w_gate, expert_w, expert_b, shared_w,
      shared_b.reshape(1, H))
    return out, loss[0, 0]


# speedup vs baseline: 1.6132x; 1.6132x over previous
import functools

import jax
import jax.numpy as jnp
from jax.experimental import pallas as pl
from jax.experimental.pallas import tpu as pltpu

B, L, D = 4, 2048, 768
E, K, H = 8, 2, 16
NSPLIT = 2
LS = L // NSPLIT


def _moe_kernel(*refs):
    x_refs = refs[:NSPLIT]
    (w_gate_ref, expert_w_ref, expert_b_ref, shared_w_ref, shared_b_ref,
     out_ref, loss_ref, gates_acc) = refs[NSPLIT:]
    b = pl.program_id(0)
    nb = pl.num_programs(0)

    gx = x_refs[0][0].sum(axis=0, keepdims=True)
    for r in x_refs[1:]:
        gx = gx + r[0].sum(axis=0, keepdims=True)
    gx = gx * (1.0 / L)
    logits = jnp.dot(gx, w_gate_ref[...],
                     preferred_element_type=jnp.float32)

    lane = jax.lax.broadcasted_iota(jnp.int32, (1, E), 1)
    m1 = jnp.max(logits)
    i1 = jnp.min(jnp.where(logits == m1, lane, E))
    mask1 = lane == i1
    l2 = jnp.where(mask1, -jnp.inf, logits)
    m2 = jnp.max(l2)
    i2 = jnp.min(jnp.where(l2 == m2, lane, E))
    mask2 = lane == i2
    t = jnp.exp(m2 - m1)
    g1 = 1.0 / (1.0 + t)
    g2 = t / (1.0 + t)
    gates_row = jnp.where(mask1, g1, 0.0) + jnp.where(mask2, g2, 0.0)

    m_w = shared_w_ref[...]
    bias = shared_b_ref[...]
    for e in range(E):
        ge = jnp.sum(jnp.where(lane == e, gates_row, 0.0))
        m_w = m_w + ge * expert_w_ref[e]
        bias = bias + ge * expert_b_ref[e][None, :]
    for i, r in enumerate(x_refs):
        yb = jax.lax.dot_general(
            r[0], m_w,
            (((1,), (0,)), ((), ())),
            preferred_element_type=jnp.float32)
        out_ref[0, i * LS:(i + 1) * LS, :] = yb + bias

    row = jax.lax.broadcasted_iota(jnp.int32, (B, E), 0)

    @pl.when(b == 0)
    def _():
        gates_acc[...] = jnp.zeros((B, E), jnp.float32)

    gates_acc[...] = jnp.where(row == b, gates_row, gates_acc[...])

    @pl.when(b == nb - 1)
    def _():
        gates_all = gates_acc[...]
        eps = 1e-10

        def cv2(v):
            mean = jnp.sum(v) * (1.0 / E)
            var = jnp.sum((v - mean) ** 2) * (1.0 / (E - 1))
            return var / (mean * mean + eps)

        importance = jnp.sum(gates_all, axis=0, keepdims=True)
        load = jnp.sum((gates_all > 0).astype(jnp.float32), axis=0,
                       keepdims=True)
        loss_ref[...] = jnp.full((1, 1), (cv2(importance) + cv2(load)) * 1e-2,
                                 jnp.float32)


@functools.partial(jax.jit, static_argnames=("interpret",))
def kernel(x, w_gate, expert_w, expert_b, shared_w, shared_b,
           interpret=False):
    x_specs = [
        pl.BlockSpec((1, LS, D), functools.partial(lambda i, b: (b, i, 0), i))
        for i in range(NSPLIT)
    ]
    out, loss = pl.pallas_call(
        _moe_kernel,
        grid=(B,),
        in_specs=x_specs + [
            pl.BlockSpec((D, E), lambda b: (0, 0)),
            pl.BlockSpec((E, D, H), lambda b: (0, 0, 0)),
            pl.BlockSpec((E, H), lambda b: (0, 0)),
            pl.BlockSpec((D, H), lambda b: (0, 0)),
            pl.BlockSpec((1, H), lambda b: (0, 0)),
        ],
        out_specs=[
            pl.BlockSpec((1, L, H), lambda b: (b, 0, 0)),
            pl.BlockSpec((1, 1), lambda b: (0, 0)),
        ],
        out_shape=[
            jax.ShapeDtypeStruct((B, L, H), jnp.float32),
            jax.ShapeDtypeStruct((1, 1), jnp.float32),
        ],
        scratch_shapes=[pltpu.VMEM((B, E), jnp.float32)],
        interpret=interpret,
    )(*([x] * NSPLIT), w_gate, expert_w, expert_b, shared_w,
      shared_b.reshape(1, H))
    return out, loss[0, 0]
